# baseline (speedup 1.0000x reference)
import functools

import jax
import jax.numpy as jnp
from jax import lax
from jax.experimental import pallas as pl
from jax.experimental.pallas import tpu as pltpu
from jax.experimental.pallas import tpu_sc as plsc

_NSLOT = 10


def _make_gather(batch, dim, nwide, nc, ns):
    nw = nc * ns
    bpw = batch // nw
    ng = bpw // 2
    nchunk = bpw // 128
    gpc = ng // nchunk
    mesh = plsc.VectorSubcoreMesh(core_axis_name="c", subcore_axis_name="s")

    @functools.partial(
        pl.kernel,
        out_type=[
            jax.ShapeDtypeStruct((batch, nwide), jnp.float32),
            jax.ShapeDtypeStruct((batch, nwide), jnp.float32),
        ],
        mesh=mesh,
        compiler_params=pltpu.CompilerParams(needs_layout_passes=False),
        scratch_types=[
            pltpu.VMEM((bpw,), jnp.int32),
            pltpu.VMEM((bpw,), jnp.int32),
            pltpu.VMEM((nchunk, 128), jnp.int32),
            pltpu.VMEM((nchunk, 128), jnp.int32),
            pltpu.VMEM((_NSLOT, dim, 128), jnp.float32),
            pltpu.VMEM((128, nwide), jnp.float32),
            pltpu.VMEM((128, nwide), jnp.float32),
            pltpu.SemaphoreType.DMA((_NSLOT,)),
            pltpu.SemaphoreType.DMA,
            pltpu.SemaphoreType.DMA,
        ],
    )
    def gather(utT_hbm, itT_hbm, su_hbm, si_hbm, pu_hbm, pi_hbm,
               ue_hbm, ie_hbm, sidu_v, sidi_v, permu_v, permi_v,
               tile_v, outu_v, outi_v, tsem, uwsem, iwsem):
        wid = lax.axis_index("s") * nc + lax.axis_index("c")
        base = wid * bpw
        lanes = lax.iota(jnp.int32, 16)
        pltpu.sync_copy(su_hbm.at[pl.ds(base, bpw)], sidu_v)
        pltpu.sync_copy(si_hbm.at[pl.ds(base, bpw)], sidi_v)
        pltpu.sync_copy(pu_hbm.at[pl.ds(wid * nchunk, nchunk)], permu_v)
        pltpu.sync_copy(pi_hbm.at[pl.ds(wid * nchunk, nchunk)], permi_v)

        def ids_of_group(sid_v, g):
            vec = sid_v[pl.ds((g // 8) * 16, 16)]
            l0 = (2 * g) % 16
            s0 = jnp.max(jnp.where(lanes == l0, vec, 0))
            s1 = jnp.max(jnp.where(lanes == l0 + 1, vec, 0))
            return s0, s1

        def fire_one(tT_hbm, sid, cur, nf):
            t = sid >> 7
            ch = (t != cur).astype(jnp.int32)

            @pl.when(ch == 1)
            def _():
                off = pl.multiple_of(t * 128, 128)
                pltpu.async_copy(tT_hbm.at[:, pl.ds(off, 128)],
                                 tile_v.at[nf % _NSLOT],
                                 tsem.at[nf % _NSLOT])

            return t, nf + ch

        def extract_one(out_v, sid, row, cur, ne, slot):
            t = sid >> 7
            ch = (t != cur).astype(jnp.int32)

            @pl.when(ch == 1)
            def _():
                pltpu.make_async_copy(utT_hbm.at[:, pl.ds(0, 128)],
                                      tile_v.at[0],
                                      tsem.at[ne % _NSLOT]).wait()

            slot = jnp.where(ch == 1, ne % _NSLOT, slot)
            sv = jnp.full((16,), slot, jnp.int32)
            c = jnp.full((16,), sid & 127, jnp.int32)
            krow = jnp.full((16,), row, jnp.int32)
            for g in range(dim // 16):
                vals = plsc.load_gather(tile_v, [sv, g * 16 + lanes, c])
                plsc.store_scatter(out_v, [krow, g * 16 + lanes], vals)
            return t, ne + ch, slot

        def group_step(j, carry, extract):
            (cuf, cif, nf, cue, cie, ne, su, si) = carry
            u0, u1 = ids_of_group(sidu_v, jnp.minimum(j + 1, ng - 1))
            i0, i1 = ids_of_group(sidi_v, jnp.minimum(j + 1, ng - 1))
            live = (j + 1 < ng).astype(jnp.int32)
            u0 = jnp.where(live == 1, u0, cuf << 7)
            i0 = jnp.where(live == 1, i0, cif << 7)
            u1 = jnp.where(live == 1, u1, cuf << 7)
            i1 = jnp.where(live == 1, i1, cif << 7)
            cuf, nf = fire_one(utT_hbm, u0, cuf, nf)
            cif, nf = fire_one(itT_hbm, i0, cif, nf)
            cuf, nf = fire_one(utT_hbm, u1, cuf, nf)
            cif, nf = fire_one(itT_hbm, i1, cif, nf)
            eu0, eu1 = ids_of_group(sidu_v, j)
            ei0, ei1 = ids_of_group(sidi_v, j)
            row = 2 * j - extract
            cue, ne, su = extract_one(outu_v, eu0, row, cue, ne, su)
            cie, ne, si = extract_one(outi_v, ei0, row, cie, ne, si)
            cue, ne, su = extract_one(outu_v, eu1, row + 1, cue, ne, su)
            cie, ne, si = extract_one(outi_v, ei1, row + 1, cie, ne, si)
            return (cuf, cif, nf, cue, cie, ne, su, si)

        carry = (jnp.int32(-1), jnp.int32(-1), jnp.int32(0),
                 jnp.int32(-1), jnp.int32(-1), jnp.int32(0),
                 jnp.int32(0), jnp.int32(0))
        u0, u1 = ids_of_group(sidu_v, 0)
        i0, i1 = ids_of_group(sidi_v, 0)
        cuf, cif, nf = carry[0], carry[1], carry[2]
        cuf, nf = fire_one(utT_hbm, u0, cuf, nf)
        cif, nf = fire_one(itT_hbm, i0, cif, nf)
        cuf, nf = fire_one(utT_hbm, u1, cuf, nf)
        cif, nf = fire_one(itT_hbm, i1, cif, nf)
        carry = (cuf, cif, nf) + carry[3:]

        for c in range(nchunk):
            if c >= 1:
                pltpu.make_async_copy(
                    outu_v, ue_hbm.at[permu_v.at[c - 1]], uwsem).wait()
                pltpu.make_async_copy(
                    outi_v, ie_hbm.at[permi_v.at[c - 1]], iwsem).wait()
            carry = lax.fori_loop(
                c * gpc, (c + 1) * gpc,
                functools.partial(group_step, extract=c * 128), carry)
            pltpu.async_copy(outu_v, ue_hbm.at[permu_v.at[c]], uwsem)
            pltpu.async_copy(outi_v, ie_hbm.at[permi_v.at[c]], iwsem)
        pltpu.make_async_copy(
            outu_v, ue_hbm.at[permu_v.at[nchunk - 1]], uwsem).wait()
        pltpu.make_async_copy(
            outi_v, ie_hbm.at[permi_v.at[nchunk - 1]], iwsem).wait()

    return gather


def _mlp_body(ue_ref, ie_ref, uf_ref, if_ref, w1_ref, b1_ref, w2_ref,
              b2_ref, out_ref):
    dim = ue_ref.shape[1] // 2
    w1 = w1_ref[...]
    h = jnp.dot(ue_ref[:, :dim], w1[:dim],
                preferred_element_type=jnp.float32)
    h += jnp.dot(ie_ref[:, :dim], w1[dim:2 * dim],
                 preferred_element_type=jnp.float32)
    h += uf_ref[...] * w1[2 * dim:2 * dim + 1]
    h += if_ref[...] * w1[2 * dim + 1:2 * dim + 2]
    h = jnp.maximum(h + b1_ref[...], 0.0)
    y = jnp.dot(h, w2_ref[...], preferred_element_type=jnp.float32) + b2_ref[...]
    out_ref[...] = jax.nn.sigmoid(y)


def _make_mlp(batch, nwide, hidden, nrows, blk):
    grid = (batch // blk,)
    row = lambda i: (i, 0)
    fixed = lambda i: (0, 0)
    return pl.pallas_call(
        _mlp_body,
        grid=grid,
        in_specs=[
            pl.BlockSpec((blk, nwide), row),
            pl.BlockSpec((blk, nwide), row),
            pl.BlockSpec((blk, 1), row),
            pl.BlockSpec((blk, 1), row),
            pl.BlockSpec((nrows, hidden), fixed),
            pl.BlockSpec((1, hidden), fixed),
            pl.BlockSpec((hidden, 1), fixed),
            pl.BlockSpec((1, 1), fixed),
        ],
        out_specs=pl.BlockSpec((blk, 1), row),
        out_shape=jax.ShapeDtypeStruct((batch, 1), jnp.float32),
    )


def kernel(user_id, item_id, user_feature, item_feature, user_table,
           item_table, W1, b1, W2, b2):
    batch = user_id.shape[0]
    dim = user_table.shape[1]
    hidden = W1.shape[1]
    info = plsc.get_sparse_core_info()
    nc, ns = info.num_cores, info.num_subcores

    pos = lax.iota(jnp.int32, batch)
    su, pu = lax.sort_key_val(user_id, pos)
    si, pi = lax.sort_key_val(item_id, pos)

    gather = _make_gather(batch, dim, 128, nc, ns)
    ue, ie = gather(user_table.T, item_table.T, su, si,
                    pu.reshape(batch // 128, 128),
                    pi.reshape(batch // 128, 128))

    mlp = _make_mlp(batch, 128, hidden, W1.shape[0], 4096)
    y = mlp(ue, ie, user_feature.reshape(batch, 1),
            item_feature.reshape(batch, 1), W1, b1.reshape(1, hidden),
            W2, b2.reshape(1, 1))
    return y.reshape(batch)
